# baseline (speedup 1.0000x reference)
import functools

import jax
import jax.numpy as jnp
from jax import lax
from jax.experimental import pallas as pl
from jax.experimental.pallas import tpu as pltpu
from jax.experimental.pallas import tpu_sc as plsc

N, E, D, H, ED = 10000, 320000, 128, 64, 16
E2 = 327680
EPS = 1e-08
W = D + ED
NC, NS, L = 2, 16, 16
NW = NC * NS
CHUNK = 128
NCHUNK = E // CHUNK
SK = 4
NSUPER = NCHUNK // SK
SKS = 2
NSUPER_S = NCHUNK // SKS
RPT = N // NS

_SC_PARAMS = pltpu.CompilerParams(use_tc_tiling_on_sc=False,
                                  needs_layout_passes=False)


def _silu(x):
    return x * jax.nn.sigmoid(x)


def _dot(a, b):
    return jnp.dot(a, b, preferred_element_type=jnp.float32)


def _prep_body(h_ref, wa_ref, wb_ref, outP_ref, outQ_ref):
    hr = h_ref[...]
    outP_ref[...] = _dot(hr, wa_ref[...]).astype(jnp.bfloat16)
    outQ_ref[...] = _dot(hr, wb_ref[...]).astype(jnp.bfloat16)


def _prep(h, We1a, We1b):
    BN = 2000
    return pl.pallas_call(
        _prep_body,
        grid=(N // BN,),
        in_specs=[
            pl.BlockSpec((BN, D), lambda i: (i, 0)),
            pl.BlockSpec((D, D), lambda i: (0, 0)),
            pl.BlockSpec((D, D), lambda i: (0, 0)),
        ],
        out_specs=[
            pl.BlockSpec((BN, D), lambda i: (i, 0)),
            pl.BlockSpec((BN, D), lambda i: (i, 0)),
        ],
        out_shape=[
            jax.ShapeDtypeStruct((N, D), jnp.bfloat16),
            jax.ShapeDtypeStruct((N, D), jnp.bfloat16),
        ],
    )(h, We1a, We1b)


def _gather(tabP, tabQ, posR, posC, rows2d, cols2d, sup0, nsup):
    chunk0 = sup0 * SK
    nch = nsup * SK
    E_h = nch * CHUNK
    E2_h = ((E_h + 2047) // 2048) * 2048
    mesh = plsc.VectorSubcoreMesh(core_axis_name="c", subcore_axis_name="s")

    @functools.partial(
        pl.kernel,
        out_type=[
            jax.ShapeDtypeStruct((E2_h, D), jnp.bfloat16),
            jax.ShapeDtypeStruct((E_h, ED), jnp.float32),
            jax.ShapeDtypeStruct((E2_h,), jnp.float32),
        ],
        mesh=mesh,
        scratch_types=[
            pltpu.VMEM((SK, CHUNK), jnp.int32),
            pltpu.VMEM((SK, CHUNK), jnp.int32),
            pltpu.VMEM((SK, CHUNK, D), jnp.bfloat16),
            pltpu.VMEM((SK, CHUNK, ED), jnp.float32),
            pltpu.VMEM((SK, CHUNK), jnp.float32),
            pltpu.SemaphoreType.DMA,
            pltpu.SemaphoreType.DMA,
        ] + [pltpu.SemaphoreType.DMA] * (2 * SK),
        compiler_params=_SC_PARAMS,
    )
    def k(tabP_h, tabQ_h, posR_h, posC_h, rows_h, cols_h, pre_h, pd_h, d2_h,
          idxr, idxc, buf, bufpos, d2b, semI, semW, *sems):
        semR = sems[:SK]
        semC = sems[SK:2 * SK]
        c = lax.axis_index("c")
        s = lax.axis_index("s")
        wid = s * NC + c
        nsuper = nsup // NW + jnp.where(wid < nsup % NW, 1, 0)
        lane = lax.iota(jnp.int32, L)

        @pl.loop(0, nsuper)
        def _(i):
            base = (wid + i * NW) * SK
            fires = []
            for j in range(SK):
                fires.append(pltpu.async_copy(
                    rows_h.at[pl.ds(chunk0 + base + j, 1)],
                    idxr.at[pl.ds(j, 1)], semI))
                fires.append(pltpu.async_copy(
                    cols_h.at[pl.ds(chunk0 + base + j, 1)],
                    idxc.at[pl.ds(j, 1)], semI))
            for dsc in fires:
                dsc.wait()
            rf = []
            for j in range(SK):
                rf.append([
                    pltpu.async_copy(tabP_h.at[idxr.at[j]], buf.at[j],
                                     semR[j]),
                    pltpu.async_copy(posR_h.at[idxr.at[j]], bufpos.at[j],
                                     semR[j]),
                ])
            cf = []
            for j in range(SK):
                for dsc in rf[j]:
                    dsc.wait()
                cf.append([
                    pltpu.async_copy(tabQ_h.at[idxc.at[j]], buf.at[j],
                                     semC[j], add=True),
                    pltpu.async_copy(posC_h.at[idxc.at[j]], bufpos.at[j],
                                     semC[j], add=True),
                ])
            fires = []
            for j in range(SK):
                kk = base + j
                for dsc in cf[j]:
                    dsc.wait()
                fires.append(pltpu.async_copy(
                    buf.at[j], pre_h.at[pl.ds(kk * CHUNK, CHUNK)], semW))
                fires.append(pltpu.async_copy(
                    bufpos.at[j], pd_h.at[pl.ds(kk * CHUNK, CHUNK)], semW))
                for g in range(CHUNK // L):
                    ids = g * L + lane
                    dx = plsc.load_gather(bufpos.at[j],
                                          [ids, jnp.full((L,), 0, jnp.int32)])
                    dy = plsc.load_gather(bufpos.at[j],
                                          [ids, jnp.full((L,), 1, jnp.int32)])
                    dz = plsc.load_gather(bufpos.at[j],
                                          [ids, jnp.full((L,), 2, jnp.int32)])
                    d2b[j, pl.ds(g * L, L)] = dx * dx + dy * dy + dz * dz
                fires.append(pltpu.async_copy(
                    d2b.at[j], d2_h.at[pl.ds(kk * CHUNK, CHUNK)], semW))
            for dsc in fires:
                dsc.wait()

    return k(tabP, tabQ, posR, posC, rows2d, cols2d)


def _edge_body(pre_ref, d2_ref, eaT_ref, we1c_ref, we1d_ref, be1_ref,
               we2_ref, be2_ref, we3_ref, be3_ref, wc1_ref, bc1_ref,
               wc2_ref, bc2_ref, wc3_ref, eo_ref, scal_ref):
    dist2 = jnp.clip(jnp.transpose(d2_ref[...][None, :], (1, 0)), EPS, 100.0)
    norm = jnp.sqrt(dist2) + EPS
    ea_term = lax.dot_general(eaT_ref[...], we1c_ref[...],
                              (((0,), (0,)), ((), ())),
                              preferred_element_type=jnp.float32)
    a1 = (pre_ref[...].astype(jnp.float32) + ea_term
          + dist2 * we1d_ref[...] + be1_ref[...])
    e1 = _silu(a1)
    e2 = _silu(_dot(e1, we2_ref[...]) + be2_ref[...])
    e = _dot(e2, we3_ref[...]) + be3_ref[...]
    c1 = _silu(_dot(e, wc1_ref[...]) + bc1_ref[...])
    c2 = _silu(_dot(c1, wc2_ref[...]) + bc2_ref[...])
    cu = jnp.clip(jnp.sum(c2 * wc3_ref[...], axis=1, keepdims=True),
                  -1.0, 1.0)
    eo_ref[...] = e
    scal_ref[...] = jnp.transpose(cu / norm, (1, 0)).reshape(scal_ref.shape)


def _edge(PRE, D2, eaT, We1c, we1d, be1, We2, be2, We3, be3,
          Wc1, bc1, Wc2, bc2, Wc3, boff):
    BE = 2048
    E2_h = PRE.shape[0]
    full = lambda i: (0, 0)
    return pl.pallas_call(
        _edge_body,
        grid=(E2_h // BE,),
        in_specs=[
            pl.BlockSpec((BE, D), lambda i: (i, 0)),
            pl.BlockSpec((BE,), lambda i: (i,)),
            pl.BlockSpec((ED, BE), lambda i: (0, i + boff)),
            pl.BlockSpec((ED, D), full),
            pl.BlockSpec((1, D), full),
            pl.BlockSpec((1, D), full),
            pl.BlockSpec((D, D), full),
            pl.BlockSpec((1, D), full),
            pl.BlockSpec((D, D), full),
            pl.BlockSpec((1, D), full),
            pl.BlockSpec((D, H), full),
            pl.BlockSpec((1, H), full),
            pl.BlockSpec((H, H), full),
            pl.BlockSpec((1, H), full),
            pl.BlockSpec((1, H), full),
        ],
        out_specs=[
            pl.BlockSpec((BE, D), lambda i: (i, 0)),
            pl.BlockSpec((BE,), lambda i: (i,)),
        ],
        out_shape=[
            jax.ShapeDtypeStruct((E2_h, D), jnp.float32),
            jax.ShapeDtypeStruct((E2_h,), jnp.float32),
        ],
    )(PRE, D2, eaT, We1c, we1d, be1, We2, be2, We3, be3,
      Wc1, bc1, Wc2, bc2, Wc3)


def _scatter(EOe, PD, SCAL, rows2d, zN, zP, chunk0, nch):
    nsup_s = nch // SKS
    mesh = plsc.VectorSubcoreMesh(core_axis_name="c", subcore_axis_name="s")

    @functools.partial(
        pl.kernel,
        out_type=[
            jax.ShapeDtypeStruct((NC, N, D), jnp.float32),
            jax.ShapeDtypeStruct((NC, N, ED), jnp.float32),
        ],
        mesh=mesh,
        scratch_types=[
            pltpu.VMEM((SKS, CHUNK), jnp.int32),
            pltpu.VMEM((SKS, CHUNK, D), jnp.float32),
            pltpu.VMEM((SKS, CHUNK, ED), jnp.float32),
            pltpu.VMEM((SKS, CHUNK), jnp.float32),
            pltpu.VMEM_SHARED((N, D), jnp.float32),
            pltpu.VMEM_SHARED((N, ED), jnp.float32),
            pltpu.SemaphoreType.DMA,
        ] + [pltpu.SemaphoreType.DMA] * SKS,
        compiler_params=_SC_PARAMS,
    )
    def k(eo_h, pd_h, scal_h, rows_h, zN_h, zP_h, outN_h, outP_h,
          idx, bufE, bufpos, sbuf, accN, accP, semS, *semR):
        c = lax.axis_index("c")
        s = lax.axis_index("s")
        wid = s * NC + c
        pltpu.sync_copy(zN_h, accN.at[pl.ds(s * RPT, RPT)])
        pltpu.sync_copy(zP_h, accP.at[pl.ds(s * RPT, RPT)])
        plsc.subcore_barrier()
        nsuper = nsup_s // NW + jnp.where(wid < nsup_s % NW, 1, 0)
        lane = lax.iota(jnp.int32, L)

        @pl.loop(0, nsuper)
        def _(i):
            base = (wid + i * NW) * SKS
            rf = []
            for j in range(SKS):
                kk = base + j
                rf.append([
                    pltpu.async_copy(rows_h.at[pl.ds(chunk0 + kk, 1)],
                                     idx.at[pl.ds(j, 1)], semR[j]),
                    pltpu.async_copy(eo_h.at[pl.ds(kk * CHUNK, CHUNK)],
                                     bufE.at[j], semR[j]),
                    pltpu.async_copy(pd_h.at[pl.ds(kk * CHUNK, CHUNK)],
                                     bufpos.at[j], semR[j]),
                    pltpu.async_copy(scal_h.at[pl.ds(kk * CHUNK, CHUNK)],
                                     sbuf.at[j], semR[j]),
                ])
            fires = []
            for j in range(SKS):
                for dsc in rf[j]:
                    dsc.wait()
                for g in range(CHUNK // L):
                    ids = g * L + lane
                    sc = sbuf[j, pl.ds(g * L, L)]
                    for comp in range(3):
                        cid = jnp.full((L,), comp, jnp.int32)
                        v = plsc.load_gather(bufpos.at[j], [ids, cid]) * sc
                        plsc.store_scatter(bufpos.at[j], [ids, cid], v)
                fires.append(pltpu.async_copy(
                    bufE.at[j], accN.at[idx.at[j]], semS, add=True))
                fires.append(pltpu.async_copy(
                    bufpos.at[j], accP.at[idx.at[j]], semS, add=True))
            for dsc in fires:
                dsc.wait()

        plsc.subcore_barrier()
        pltpu.sync_copy(accN.at[pl.ds(s * RPT, RPT)],
                        outN_h.at[c, pl.ds(s * RPT, RPT)])
        pltpu.sync_copy(accP.at[pl.ds(s * RPT, RPT)],
                        outP_h.at[c, pl.ds(s * RPT, RPT)])

    return k(EOe, PD, SCAL, rows2d, zN, zP)


def _node_body(h_ref, pp_ref, an0, an1, an2, an3, ap0, ap1, ap2, ap3,
               wn1_ref, bn1_ref, wn2_ref, bn2_ref, wn3_ref, bn3_ref,
               hout_ref, pout_ref):
    aggn = ((an0[0] + an0[1]) + (an1[0] + an1[1])
            + (an2[0] + an2[1]) + (an3[0] + an3[1]))
    x = jnp.concatenate([h_ref[...], aggn], axis=1)
    n1 = _silu(_dot(x, wn1_ref[...]) + bn1_ref[...])
    n2 = _silu(_dot(n1, wn2_ref[...]) + bn2_ref[...])
    hn = _dot(n2, wn3_ref[...]) + bn3_ref[...]
    hout_ref[...] = h_ref[...] + hn
    pout_ref[...] = (pp_ref[...] + (ap0[0] + ap0[1]) + (ap1[0] + ap1[1])
                     + (ap2[0] + ap2[1]) + (ap3[0] + ap3[1]))


def _node(h, pos_pad, ACCN, ACCP, Wn1, bn1, Wn2, bn2, Wn3, bn3):
    BN = 2000
    full = lambda i: (0, 0)
    return pl.pallas_call(
        _node_body,
        grid=(N // BN,),
        in_specs=[
            pl.BlockSpec((BN, D), lambda i: (i, 0)),
            pl.BlockSpec((BN, ED), lambda i: (i, 0)),
        ] + [pl.BlockSpec((NC, BN, D), lambda i: (0, i, 0))
             for _ in range(4)]
        + [pl.BlockSpec((NC, BN, ED), lambda i: (0, i, 0))
           for _ in range(4)] + [
            pl.BlockSpec((2 * D, D), full),
            pl.BlockSpec((1, D), full),
            pl.BlockSpec((D, D), full),
            pl.BlockSpec((1, D), full),
            pl.BlockSpec((D, D), full),
            pl.BlockSpec((1, D), full),
        ],
        out_specs=[
            pl.BlockSpec((BN, D), lambda i: (i, 0)),
            pl.BlockSpec((BN, ED), lambda i: (i, 0)),
        ],
        out_shape=[
            jax.ShapeDtypeStruct((N, D), jnp.float32),
            jax.ShapeDtypeStruct((N, ED), jnp.float32),
        ],
    )(h, pos_pad, *ACCN, *ACCP, Wn1, bn1, Wn2, bn2, Wn3, bn3)


def kernel(h, edge_index, edge_attr, pos, We1, be1, We2, be2, We3, be3,
           Wc1, bc1, Wc2, bc2, Wc3, Wn1, bn1, Wn2, bn2, Wn3, bn3):
    rows2d = edge_index[0].reshape(NCHUNK, CHUNK)
    cols2d = edge_index[1].reshape(NCHUNK, CHUNK)
    eaT = jnp.pad(edge_attr.T, ((0, 0), (0, E2 - E)))
    We1a = We1[:D]
    We1b = We1[D:2 * D]
    We1c = We1[2 * D:2 * D + ED]
    we1d = We1[2 * D + ED:]
    pos_pad = jnp.pad(pos, ((0, 0), (0, ED - 3)))

    tabP, tabQ = _prep(h, We1a, We1b)
    negpos = -pos_pad
    sups = [156, 156, 156, 157]
    ew = [be1.reshape(1, D), We2, be2.reshape(1, D), We3, be3.reshape(1, D),
          Wc1, bc1.reshape(1, H), Wc2, bc2.reshape(1, H), Wc3.reshape(1, H)]
    zN = jnp.zeros((RPT, D), jnp.float32)
    zP = jnp.zeros((RPT, ED), jnp.float32)
    gs = []
    s0 = 0
    for ns in sups:
        gs.append((s0, ns, _gather(tabP, tabQ, pos_pad, negpos,
                                   rows2d, cols2d, s0, ns)))
        s0 += ns
    es = []
    for s0, ns, (PREi, PDi, D2i) in gs:
        EOi, SCi = _edge(PREi, D2i, eaT, We1c, we1d, *ew,
                         boff=s0 * SK * CHUNK // 2048)
        es.append((s0, ns, PDi, EOi, SCi))
    accs = []
    for s0, ns, PDi, EOi, SCi in es:
        accs.append(_scatter(EOi, PDi, SCi, rows2d, zN, zP,
                             s0 * SK, ns * SK))
    ACCN = [a[0] for a in accs]
    ACCP = [a[1] for a in accs]
    h_new, pos_out = _node(h, pos_pad, ACCN, ACCP,
                           Wn1, bn1.reshape(1, D), Wn2, bn2.reshape(1, D),
                           Wn3, bn3.reshape(1, D))
    return h_new, pos_out[:, :3]


# speedup vs baseline: 1.4112x; 1.4112x over previous
import functools

import jax
import jax.numpy as jnp
from jax import lax
from jax.experimental import pallas as pl
from jax.experimental.pallas import tpu as pltpu
from jax.experimental.pallas import tpu_sc as plsc

N, E, D, H, ED = 10000, 320000, 128, 64, 16
E2 = 327680
EPS = 1e-08
W = D + ED
NC, NS, L = 2, 16, 16
NW = NC * NS
CHUNK = 128
NCHUNK = E // CHUNK
SK = 4
NSUPER = NCHUNK // SK
SKS = 2
NSUPER_S = NCHUNK // SKS
RPT = N // NS

_SC_PARAMS = pltpu.CompilerParams(use_tc_tiling_on_sc=False,
                                  needs_layout_passes=False)


def _silu(x):
    return x * jax.nn.sigmoid(x)


def _dot(a, b):
    return jnp.dot(a, b, preferred_element_type=jnp.float32)


def _prep_body(h_ref, pp_ref, wa_ref, wb_ref, outR_ref, outC_ref):
    hr = h_ref[...]
    pp = pp_ref[...]
    outR_ref[...] = jnp.concatenate([_dot(hr, wa_ref[...]), pp], axis=1)
    outC_ref[...] = jnp.concatenate([_dot(hr, wb_ref[...]), -pp], axis=1)


def _prep(h, pos_pad, We1a, We1b):
    BN = 2000
    return pl.pallas_call(
        _prep_body,
        grid=(N // BN,),
        in_specs=[
            pl.BlockSpec((BN, D), lambda i: (i, 0)),
            pl.BlockSpec((BN, ED), lambda i: (i, 0)),
            pl.BlockSpec((D, D), lambda i: (0, 0)),
            pl.BlockSpec((D, D), lambda i: (0, 0)),
        ],
        out_specs=[
            pl.BlockSpec((BN, W), lambda i: (i, 0)),
            pl.BlockSpec((BN, W), lambda i: (i, 0)),
        ],
        out_shape=[
            jax.ShapeDtypeStruct((N, W), jnp.float32),
            jax.ShapeDtypeStruct((N, W), jnp.float32),
        ],
    )(h, pos_pad, We1a, We1b)


def _gather(tabR, tabC, rows2d, cols2d, sup0, nsup):
    chunk0 = sup0 * SK
    nch = nsup * SK
    E_h = nch * CHUNK
    E2_h = ((E_h + 2047) // 2048) * 2048
    mesh = plsc.VectorSubcoreMesh(core_axis_name="c", subcore_axis_name="s")

    @functools.partial(
        pl.kernel,
        out_type=[
            jax.ShapeDtypeStruct((E2_h, D), jnp.float32),
            jax.ShapeDtypeStruct((E_h, ED), jnp.float32),
            jax.ShapeDtypeStruct((E2_h,), jnp.float32),
        ],
        mesh=mesh,
        scratch_types=[
            pltpu.VMEM((SK, CHUNK), jnp.int32),
            pltpu.VMEM((SK, CHUNK), jnp.int32),
            pltpu.VMEM((SK, CHUNK, W), jnp.float32),
            pltpu.VMEM((SK, CHUNK), jnp.float32),
            pltpu.SemaphoreType.DMA,
            pltpu.SemaphoreType.DMA,
        ] + [pltpu.SemaphoreType.DMA] * (2 * SK),
        compiler_params=_SC_PARAMS,
    )
    def k(tabR_h, tabC_h, rows_h, cols_h, pre_h, pd_h, d2_h,
          idxr, idxc, buf, d2b, semI, semW, *sems):
        semR = sems[:SK]
        semC = sems[SK:2 * SK]
        c = lax.axis_index("c")
        s = lax.axis_index("s")
        wid = s * NC + c
        nsuper = nsup // NW + jnp.where(wid < nsup % NW, 1, 0)
        lane = lax.iota(jnp.int32, L)

        @pl.loop(0, nsuper)
        def _(i):
            base = (wid + i * NW) * SK
            fires = []
            for j in range(SK):
                fires.append(pltpu.async_copy(
                    rows_h.at[pl.ds(chunk0 + base + j, 1)],
                    idxr.at[pl.ds(j, 1)], semI))
                fires.append(pltpu.async_copy(
                    cols_h.at[pl.ds(chunk0 + base + j, 1)],
                    idxc.at[pl.ds(j, 1)], semI))
            for dsc in fires:
                dsc.wait()
            rf = [pltpu.async_copy(tabR_h.at[idxr.at[j]], buf.at[j], semR[j])
                  for j in range(SK)]
            cf = []
            for j in range(SK):
                rf[j].wait()
                cf.append(pltpu.async_copy(tabC_h.at[idxc.at[j]], buf.at[j],
                                           semC[j], add=True))
            fires = []
            for j in range(SK):
                kk = base + j
                cf[j].wait()
                fires.append(pltpu.async_copy(
                    buf.at[j, :, pl.ds(0, D)],
                    pre_h.at[pl.ds(kk * CHUNK, CHUNK)], semW))
                fires.append(pltpu.async_copy(
                    buf.at[j, :, pl.ds(D, ED)],
                    pd_h.at[pl.ds(kk * CHUNK, CHUNK)], semW))
                for g in range(CHUNK // L):
                    ids = g * L + lane
                    dx = plsc.load_gather(buf.at[j],
                                          [ids, jnp.full((L,), D, jnp.int32)])
                    dy = plsc.load_gather(buf.at[j],
                                          [ids, jnp.full((L,), D + 1, jnp.int32)])
                    dz = plsc.load_gather(buf.at[j],
                                          [ids, jnp.full((L,), D + 2, jnp.int32)])
                    d2b[j, pl.ds(g * L, L)] = dx * dx + dy * dy + dz * dz
                fires.append(pltpu.async_copy(
                    d2b.at[j], d2_h.at[pl.ds(kk * CHUNK, CHUNK)], semW))
            for dsc in fires:
                dsc.wait()

    return k(tabR, tabC, rows2d, cols2d)


def _edge_body(pre_ref, d2_ref, eaT_ref, we1c_ref, we1d_ref, be1_ref,
               we2_ref, be2_ref, we3_ref, be3_ref, wc1_ref, bc1_ref,
               wc2_ref, bc2_ref, wc3_ref, eo_ref, scal_ref):
    dist2 = jnp.clip(jnp.transpose(d2_ref[...][None, :], (1, 0)), EPS, 100.0)
    norm = jnp.sqrt(dist2) + EPS
    ea_term = lax.dot_general(eaT_ref[...], we1c_ref[...],
                              (((0,), (0,)), ((), ())),
                              preferred_element_type=jnp.float32)
    a1 = pre_ref[...] + ea_term + dist2 * we1d_ref[...] + be1_ref[...]
    e1 = _silu(a1)
    e2 = _silu(_dot(e1, we2_ref[...]) + be2_ref[...])
    e = _dot(e2, we3_ref[...]) + be3_ref[...]
    c1 = _silu(_dot(e, wc1_ref[...]) + bc1_ref[...])
    c2 = _silu(_dot(c1, wc2_ref[...]) + bc2_ref[...])
    cu = jnp.clip(jnp.sum(c2 * wc3_ref[...], axis=1, keepdims=True),
                  -1.0, 1.0)
    eo_ref[...] = e
    scal_ref[...] = jnp.transpose(cu / norm, (1, 0)).reshape(scal_ref.shape)


def _edge(PRE, D2, eaT, We1c, we1d, be1, We2, be2, We3, be3,
          Wc1, bc1, Wc2, bc2, Wc3, boff):
    BE = 2048
    E2_h = PRE.shape[0]
    full = lambda i: (0, 0)
    return pl.pallas_call(
        _edge_body,
        grid=(E2_h // BE,),
        in_specs=[
            pl.BlockSpec((BE, D), lambda i: (i, 0)),
            pl.BlockSpec((BE,), lambda i: (i,)),
            pl.BlockSpec((ED, BE), lambda i: (0, i + boff)),
            pl.BlockSpec((ED, D), full),
            pl.BlockSpec((1, D), full),
            pl.BlockSpec((1, D), full),
            pl.BlockSpec((D, D), full),
            pl.BlockSpec((1, D), full),
            pl.BlockSpec((D, D), full),
            pl.BlockSpec((1, D), full),
            pl.BlockSpec((D, H), full),
            pl.BlockSpec((1, H), full),
            pl.BlockSpec((H, H), full),
            pl.BlockSpec((1, H), full),
            pl.BlockSpec((1, H), full),
        ],
        out_specs=[
            pl.BlockSpec((BE, D), lambda i: (i, 0)),
            pl.BlockSpec((BE,), lambda i: (i,)),
        ],
        out_shape=[
            jax.ShapeDtypeStruct((E2_h, D), jnp.float32),
            jax.ShapeDtypeStruct((E2_h,), jnp.float32),
        ],
    )(PRE, D2, eaT, We1c, we1d, be1, We2, be2, We3, be3,
      Wc1, bc1, Wc2, bc2, Wc3)


def _scatter(EOe, PD, SCAL, rows2d, zN, zP, chunk0, nch):
    nsup_s = nch // SKS
    mesh = plsc.VectorSubcoreMesh(core_axis_name="c", subcore_axis_name="s")

    @functools.partial(
        pl.kernel,
        out_type=[
            jax.ShapeDtypeStruct((NC, N, D), jnp.float32),
            jax.ShapeDtypeStruct((NC, N, ED), jnp.float32),
        ],
        mesh=mesh,
        scratch_types=[
            pltpu.VMEM((SKS, CHUNK), jnp.int32),
            pltpu.VMEM((SKS, CHUNK, D), jnp.float32),
            pltpu.VMEM((SKS, CHUNK, ED), jnp.float32),
            pltpu.VMEM((SKS, CHUNK), jnp.float32),
            pltpu.VMEM_SHARED((N, D), jnp.float32),
            pltpu.VMEM_SHARED((N, ED), jnp.float32),
            pltpu.SemaphoreType.DMA,
        ] + [pltpu.SemaphoreType.DMA] * SKS,
        compiler_params=_SC_PARAMS,
    )
    def k(eo_h, pd_h, scal_h, rows_h, zN_h, zP_h, outN_h, outP_h,
          idx, bufE, bufpos, sbuf, accN, accP, semS, *semR):
        c = lax.axis_index("c")
        s = lax.axis_index("s")
        wid = s * NC + c
        pltpu.sync_copy(zN_h, accN.at[pl.ds(s * RPT, RPT)])
        pltpu.sync_copy(zP_h, accP.at[pl.ds(s * RPT, RPT)])
        plsc.subcore_barrier()
        nsuper = nsup_s // NW + jnp.where(wid < nsup_s % NW, 1, 0)
        lane = lax.iota(jnp.int32, L)

        @pl.loop(0, nsuper)
        def _(i):
            base = (wid + i * NW) * SKS
            rf = []
            for j in range(SKS):
                kk = base + j
                rf.append([
                    pltpu.async_copy(rows_h.at[pl.ds(chunk0 + kk, 1)],
                                     idx.at[pl.ds(j, 1)], semR[j]),
                    pltpu.async_copy(eo_h.at[pl.ds(kk * CHUNK, CHUNK)],
                                     bufE.at[j], semR[j]),
                    pltpu.async_copy(pd_h.at[pl.ds(kk * CHUNK, CHUNK)],
                                     bufpos.at[j], semR[j]),
                    pltpu.async_copy(scal_h.at[pl.ds(kk * CHUNK, CHUNK)],
                                     sbuf.at[j], semR[j]),
                ])
            fires = []
            for j in range(SKS):
                for dsc in rf[j]:
                    dsc.wait()
                for g in range(CHUNK // L):
                    ids = g * L + lane
                    sc = sbuf[j, pl.ds(g * L, L)]
                    for comp in range(3):
                        cid = jnp.full((L,), comp, jnp.int32)
                        v = plsc.load_gather(bufpos.at[j], [ids, cid]) * sc
                        plsc.store_scatter(bufpos.at[j], [ids, cid], v)
                fires.append(pltpu.async_copy(
                    bufE.at[j], accN.at[idx.at[j]], semS, add=True))
                fires.append(pltpu.async_copy(
                    bufpos.at[j], accP.at[idx.at[j]], semS, add=True))
            for dsc in fires:
                dsc.wait()

        plsc.subcore_barrier()
        pltpu.sync_copy(accN.at[pl.ds(s * RPT, RPT)],
                        outN_h.at[c, pl.ds(s * RPT, RPT)])
        pltpu.sync_copy(accP.at[pl.ds(s * RPT, RPT)],
                        outP_h.at[c, pl.ds(s * RPT, RPT)])

    return k(EOe, PD, SCAL, rows2d, zN, zP)


NSLICE = 2


def _node_body(h_ref, pp_ref, *refs):
    ans = refs[:NSLICE]
    aps = refs[NSLICE:2 * NSLICE]
    (wn1_ref, bn1_ref, wn2_ref, bn2_ref, wn3_ref, bn3_ref,
     hout_ref, pout_ref) = refs[2 * NSLICE:]
    aggn = sum((a[0] + a[1] for a in ans[1:]), ans[0][0] + ans[0][1])
    x = jnp.concatenate([h_ref[...], aggn], axis=1)
    n1 = _silu(_dot(x, wn1_ref[...]) + bn1_ref[...])
    n2 = _silu(_dot(n1, wn2_ref[...]) + bn2_ref[...])
    hn = _dot(n2, wn3_ref[...]) + bn3_ref[...]
    hout_ref[...] = h_ref[...] + hn
    pout_ref[...] = pp_ref[...] + sum(
        (a[0] + a[1] for a in aps[1:]), aps[0][0] + aps[0][1])


def _node(h, pos_pad, ACCN, ACCP, Wn1, bn1, Wn2, bn2, Wn3, bn3):
    BN = 2000
    full = lambda i: (0, 0)
    return pl.pallas_call(
        _node_body,
        grid=(N // BN,),
        in_specs=[
            pl.BlockSpec((BN, D), lambda i: (i, 0)),
            pl.BlockSpec((BN, ED), lambda i: (i, 0)),
        ] + [pl.BlockSpec((NC, BN, D), lambda i: (0, i, 0))
             for _ in range(NSLICE)]
        + [pl.BlockSpec((NC, BN, ED), lambda i: (0, i, 0))
           for _ in range(NSLICE)] + [
            pl.BlockSpec((2 * D, D), full),
            pl.BlockSpec((1, D), full),
            pl.BlockSpec((D, D), full),
            pl.BlockSpec((1, D), full),
            pl.BlockSpec((D, D), full),
            pl.BlockSpec((1, D), full),
        ],
        out_specs=[
            pl.BlockSpec((BN, D), lambda i: (i, 0)),
            pl.BlockSpec((BN, ED), lambda i: (i, 0)),
        ],
        out_shape=[
            jax.ShapeDtypeStruct((N, D), jnp.float32),
            jax.ShapeDtypeStruct((N, ED), jnp.float32),
        ],
    )(h, pos_pad, *ACCN, *ACCP, Wn1, bn1, Wn2, bn2, Wn3, bn3)


def kernel(h, edge_index, edge_attr, pos, We1, be1, We2, be2, We3, be3,
           Wc1, bc1, Wc2, bc2, Wc3, Wn1, bn1, Wn2, bn2, Wn3, bn3):
    rows2d = edge_index[0].reshape(NCHUNK, CHUNK)
    cols2d = edge_index[1].reshape(NCHUNK, CHUNK)
    eaT = jnp.pad(edge_attr.T, ((0, 0), (0, E2 - E)))
    We1a = We1[:D]
    We1b = We1[D:2 * D]
    We1c = We1[2 * D:2 * D + ED]
    we1d = We1[2 * D + ED:]
    pos_pad = jnp.pad(pos, ((0, 0), (0, ED - 3)))

    tabR, tabC = _prep(h, pos_pad, We1a, We1b)
    sups = [312, 313] if NSLICE == 2 else [156, 156, 156, 157]
    ew = [be1.reshape(1, D), We2, be2.reshape(1, D), We3, be3.reshape(1, D),
          Wc1, bc1.reshape(1, H), Wc2, bc2.reshape(1, H), Wc3.reshape(1, H)]
    zN = jnp.zeros((RPT, D), jnp.float32)
    zP = jnp.zeros((RPT, ED), jnp.float32)
    gs = []
    s0 = 0
    for ns in sups:
        gs.append((s0, ns, _gather(tabR, tabC, rows2d, cols2d, s0, ns)))
        s0 += ns
    es = []
    for s0, ns, (PREi, PDi, D2i) in gs:
        EOi, SCi = _edge(PREi, D2i, eaT, We1c, we1d, *ew,
                         boff=s0 * SK * CHUNK // 2048)
        es.append((s0, ns, PDi, EOi, SCi))
    accs = []
    for s0, ns, PDi, EOi, SCi in es:
        accs.append(_scatter(EOi, PDi, SCi, rows2d, zN, zP,
                             s0 * SK, ns * SK))
    ACCN = [a[0] for a in accs]
    ACCP = [a[1] for a in accs]
    h_new, pos_out = _node(h, pos_pad, ACCN, ACCP,
                           Wn1, bn1.reshape(1, D), Wn2, bn2.reshape(1, D),
                           Wn3, bn3.reshape(1, D))
    return h_new, pos_out[:, :3]


# speedup vs baseline: 1.4415x; 1.0215x over previous
import functools

import jax
import jax.numpy as jnp
from jax import lax
from jax.experimental import pallas as pl
from jax.experimental.pallas import tpu as pltpu
from jax.experimental.pallas import tpu_sc as plsc

N, E, D, H, ED = 10000, 320000, 128, 64, 16
E2 = 327680
EPS = 1e-08
W = D + ED
NC, NS, L = 2, 16, 16
NW = NC * NS
CHUNK = 128
NCHUNK = E // CHUNK
SK = 4
NSUPER = NCHUNK // SK
SKS = 2
NSUPER_S = NCHUNK // SKS
RPT = N // NS

_SC_PARAMS = pltpu.CompilerParams(use_tc_tiling_on_sc=False,
                                  needs_layout_passes=False)


def _silu(x):
    return x * jax.nn.sigmoid(x)


def _dot(a, b):
    return jnp.dot(a, b, preferred_element_type=jnp.float32)


def _prep_body(h_ref, pp_ref, wa_ref, wb_ref, outR_ref, outC_ref):
    hr = h_ref[...]
    pp = pp_ref[...]
    outR_ref[...] = jnp.concatenate([_dot(hr, wa_ref[...]), pp], axis=1)
    outC_ref[...] = jnp.concatenate([_dot(hr, wb_ref[...]), -pp], axis=1)


def _prep(h, pos_pad, We1a, We1b):
    BN = 2000
    return pl.pallas_call(
        _prep_body,
        grid=(N // BN,),
        in_specs=[
            pl.BlockSpec((BN, D), lambda i: (i, 0)),
            pl.BlockSpec((BN, ED), lambda i: (i, 0)),
            pl.BlockSpec((D, D), lambda i: (0, 0)),
            pl.BlockSpec((D, D), lambda i: (0, 0)),
        ],
        out_specs=[
            pl.BlockSpec((BN, W), lambda i: (i, 0)),
            pl.BlockSpec((BN, W), lambda i: (i, 0)),
        ],
        out_shape=[
            jax.ShapeDtypeStruct((N, W), jnp.float32),
            jax.ShapeDtypeStruct((N, W), jnp.float32),
        ],
    )(h, pos_pad, We1a, We1b)


def _gather(tabR, tabC, rows2d, cols2d, sup0, nsup):
    chunk0 = sup0 * SK
    nch = nsup * SK
    E_h = nch * CHUNK
    E2_h = ((E_h + 2047) // 2048) * 2048
    mesh = plsc.VectorSubcoreMesh(core_axis_name="c", subcore_axis_name="s")

    @functools.partial(
        pl.kernel,
        out_type=[
            jax.ShapeDtypeStruct((E2_h, D), jnp.float32),
            jax.ShapeDtypeStruct((E_h, ED), jnp.float32),
            jax.ShapeDtypeStruct((E2_h,), jnp.float32),
        ],
        mesh=mesh,
        scratch_types=[
            pltpu.VMEM((SK, CHUNK), jnp.int32),
            pltpu.VMEM((SK, CHUNK), jnp.int32),
            pltpu.VMEM((SK, CHUNK, W), jnp.float32),
            pltpu.VMEM((SK, CHUNK), jnp.float32),
            pltpu.SemaphoreType.DMA,
            pltpu.SemaphoreType.DMA,
        ] + [pltpu.SemaphoreType.DMA] * (2 * SK),
        compiler_params=_SC_PARAMS,
    )
    def k(tabR_h, tabC_h, rows_h, cols_h, pre_h, pd_h, d2_h,
          idxr, idxc, buf, d2b, semI, semW, *sems):
        semR = sems[:SK]
        semC = sems[SK:2 * SK]
        c = lax.axis_index("c")
        s = lax.axis_index("s")
        wid = s * NC + c
        nsuper = nsup // NW + jnp.where(wid < nsup % NW, 1, 0)
        lane = lax.iota(jnp.int32, L)

        @pl.loop(0, nsuper)
        def _(i):
            base = (wid + i * NW) * SK
            fires = []
            for j in range(SK):
                fires.append(pltpu.async_copy(
                    rows_h.at[pl.ds(chunk0 + base + j, 1)],
                    idxr.at[pl.ds(j, 1)], semI))
                fires.append(pltpu.async_copy(
                    cols_h.at[pl.ds(chunk0 + base + j, 1)],
                    idxc.at[pl.ds(j, 1)], semI))
            for dsc in fires:
                dsc.wait()
            rf = [pltpu.async_copy(tabR_h.at[idxr.at[j]], buf.at[j], semR[j])
                  for j in range(SK)]
            cf = []
            for j in range(SK):
                rf[j].wait()
                cf.append(pltpu.async_copy(tabC_h.at[idxc.at[j]], buf.at[j],
                                           semC[j], add=True))
            fires = []
            for j in range(SK):
                kk = base + j
                cf[j].wait()
                fires.append(pltpu.async_copy(
                    buf.at[j, :, pl.ds(0, D)],
                    pre_h.at[pl.ds(kk * CHUNK, CHUNK)], semW))
                fires.append(pltpu.async_copy(
                    buf.at[j, :, pl.ds(D, ED)],
                    pd_h.at[pl.ds(kk * CHUNK, CHUNK)], semW))
                for g in range(CHUNK // L):
                    ids = g * L + lane
                    dx = plsc.load_gather(buf.at[j],
                                          [ids, jnp.full((L,), D, jnp.int32)])
                    dy = plsc.load_gather(buf.at[j],
                                          [ids, jnp.full((L,), D + 1, jnp.int32)])
                    dz = plsc.load_gather(buf.at[j],
                                          [ids, jnp.full((L,), D + 2, jnp.int32)])
                    d2b[j, pl.ds(g * L, L)] = dx * dx + dy * dy + dz * dz
                fires.append(pltpu.async_copy(
                    d2b.at[j], d2_h.at[pl.ds(kk * CHUNK, CHUNK)], semW))
            for dsc in fires:
                dsc.wait()

    return k(tabR, tabC, rows2d, cols2d)


def _edge_body(pre_ref, d2_ref, eaT_ref, we1c_ref, we1d_ref, be1_ref,
               we2_ref, be2_ref, we3_ref, be3_ref, wc1_ref, bc1_ref,
               wc2_ref, bc2_ref, wc3_ref, eo_ref, scal_ref):
    dist2 = jnp.clip(jnp.transpose(d2_ref[...][None, :], (1, 0)), EPS, 100.0)
    norm = jnp.sqrt(dist2) + EPS
    ea_term = lax.dot_general(eaT_ref[...], we1c_ref[...],
                              (((0,), (0,)), ((), ())),
                              preferred_element_type=jnp.float32)
    a1 = pre_ref[...] + ea_term + dist2 * we1d_ref[...] + be1_ref[...]
    e1 = _silu(a1)
    e2 = _silu(_dot(e1, we2_ref[...]) + be2_ref[...])
    e = _dot(e2, we3_ref[...]) + be3_ref[...]
    c1 = _silu(_dot(e, wc1_ref[...]) + bc1_ref[...])
    c2 = _silu(_dot(c1, wc2_ref[...]) + bc2_ref[...])
    cu = jnp.clip(jnp.sum(c2 * wc3_ref[...], axis=1, keepdims=True),
                  -1.0, 1.0)
    eo_ref[...] = e
    scal_ref[...] = jnp.transpose(cu / norm, (1, 0)).reshape(scal_ref.shape)


def _edge(PRE, D2, eaT, We1c, we1d, be1, We2, be2, We3, be3,
          Wc1, bc1, Wc2, bc2, Wc3, boff):
    BE = 2048
    E2_h = PRE.shape[0]
    full = lambda i: (0, 0)
    return pl.pallas_call(
        _edge_body,
        grid=(E2_h // BE,),
        in_specs=[
            pl.BlockSpec((BE, D), lambda i: (i, 0)),
            pl.BlockSpec((BE,), lambda i: (i,)),
            pl.BlockSpec((ED, BE), lambda i: (0, i + boff)),
            pl.BlockSpec((ED, D), full),
            pl.BlockSpec((1, D), full),
            pl.BlockSpec((1, D), full),
            pl.BlockSpec((D, D), full),
            pl.BlockSpec((1, D), full),
            pl.BlockSpec((D, D), full),
            pl.BlockSpec((1, D), full),
            pl.BlockSpec((D, H), full),
            pl.BlockSpec((1, H), full),
            pl.BlockSpec((H, H), full),
            pl.BlockSpec((1, H), full),
            pl.BlockSpec((1, H), full),
        ],
        out_specs=[
            pl.BlockSpec((BE, D), lambda i: (i, 0)),
            pl.BlockSpec((BE,), lambda i: (i,)),
        ],
        out_shape=[
            jax.ShapeDtypeStruct((E2_h, D), jnp.float32),
            jax.ShapeDtypeStruct((E2_h,), jnp.float32),
        ],
    )(PRE, D2, eaT, We1c, we1d, be1, We2, be2, We3, be3,
      Wc1, bc1, Wc2, bc2, Wc3)


def _scatter(EOe, PD, SCAL, rows2d, zN, zP, chunk0, nch):
    nsup_s = nch // SKS
    mesh = plsc.VectorSubcoreMesh(core_axis_name="c", subcore_axis_name="s")

    @functools.partial(
        pl.kernel,
        out_type=[
            jax.ShapeDtypeStruct((NC, N, D), jnp.float32),
            jax.ShapeDtypeStruct((NC, N, ED), jnp.float32),
        ],
        mesh=mesh,
        scratch_types=[
            pltpu.VMEM((SKS, CHUNK), jnp.int32),
            pltpu.VMEM((SKS, CHUNK, D), jnp.float32),
            pltpu.VMEM((SKS, CHUNK, ED), jnp.float32),
            pltpu.VMEM((SKS, CHUNK), jnp.float32),
            pltpu.VMEM_SHARED((N, D), jnp.float32),
            pltpu.VMEM_SHARED((N, ED), jnp.float32),
            pltpu.SemaphoreType.DMA,
        ] + [pltpu.SemaphoreType.DMA] * SKS,
        compiler_params=_SC_PARAMS,
    )
    def k(eo_h, pd_h, scal_h, rows_h, zN_h, zP_h, outN_h, outP_h,
          idx, bufE, bufpos, sbuf, accN, accP, semS, *semR):
        c = lax.axis_index("c")
        s = lax.axis_index("s")
        wid = s * NC + c
        pltpu.sync_copy(zN_h, accN.at[pl.ds(s * RPT, RPT)])
        pltpu.sync_copy(zP_h, accP.at[pl.ds(s * RPT, RPT)])
        plsc.subcore_barrier()
        nsuper = nsup_s // NW + jnp.where(wid < nsup_s % NW, 1, 0)
        lane = lax.iota(jnp.int32, L)

        @pl.loop(0, nsuper)
        def _(i):
            base = (wid + i * NW) * SKS
            rf = []
            for j in range(SKS):
                kk = base + j
                rf.append([
                    pltpu.async_copy(rows_h.at[pl.ds(chunk0 + kk, 1)],
                                     idx.at[pl.ds(j, 1)], semR[j]),
                    pltpu.async_copy(eo_h.at[pl.ds(kk * CHUNK, CHUNK)],
                                     bufE.at[j], semR[j]),
                    pltpu.async_copy(pd_h.at[pl.ds(kk * CHUNK, CHUNK)],
                                     bufpos.at[j], semR[j]),
                    pltpu.async_copy(scal_h.at[pl.ds(kk * CHUNK, CHUNK)],
                                     sbuf.at[j], semR[j]),
                ])
            fires = []
            for j in range(SKS):
                for dsc in rf[j]:
                    dsc.wait()
                for g in range(CHUNK // L):
                    ids = g * L + lane
                    sc = sbuf[j, pl.ds(g * L, L)]
                    for comp in range(3):
                        cid = jnp.full((L,), comp, jnp.int32)
                        v = plsc.load_gather(bufpos.at[j], [ids, cid]) * sc
                        plsc.store_scatter(bufpos.at[j], [ids, cid], v)
                fires.append(pltpu.async_copy(
                    bufE.at[j], accN.at[idx.at[j]], semS, add=True))
                fires.append(pltpu.async_copy(
                    bufpos.at[j], accP.at[idx.at[j]], semS, add=True))
            for dsc in fires:
                dsc.wait()

        plsc.subcore_barrier()
        pltpu.sync_copy(accN.at[pl.ds(s * RPT, RPT)],
                        outN_h.at[c, pl.ds(s * RPT, RPT)])
        pltpu.sync_copy(accP.at[pl.ds(s * RPT, RPT)],
                        outP_h.at[c, pl.ds(s * RPT, RPT)])

    return k(EOe, PD, SCAL, rows2d, zN, zP)


NSLICE = 4


def _node_body(h_ref, pp_ref, *refs):
    ans = refs[:NSLICE]
    aps = refs[NSLICE:2 * NSLICE]
    (wn1_ref, bn1_ref, wn2_ref, bn2_ref, wn3_ref, bn3_ref,
     hout_ref, pout_ref) = refs[2 * NSLICE:]
    aggn = sum((a[0] + a[1] for a in ans[1:]), ans[0][0] + ans[0][1])
    x = jnp.concatenate([h_ref[...], aggn], axis=1)
    n1 = _silu(_dot(x, wn1_ref[...]) + bn1_ref[...])
    n2 = _silu(_dot(n1, wn2_ref[...]) + bn2_ref[...])
    hn = _dot(n2, wn3_ref[...]) + bn3_ref[...]
    hout_ref[...] = h_ref[...] + hn
    pout_ref[...] = pp_ref[...] + sum(
        (a[0] + a[1] for a in aps[1:]), aps[0][0] + aps[0][1])


def _node(h, pos_pad, ACCN, ACCP, Wn1, bn1, Wn2, bn2, Wn3, bn3):
    BN = 2000
    full = lambda i: (0, 0)
    return pl.pallas_call(
        _node_body,
        grid=(N // BN,),
        in_specs=[
            pl.BlockSpec((BN, D), lambda i: (i, 0)),
            pl.BlockSpec((BN, ED), lambda i: (i, 0)),
        ] + [pl.BlockSpec((NC, BN, D), lambda i: (0, i, 0))
             for _ in range(NSLICE)]
        + [pl.BlockSpec((NC, BN, ED), lambda i: (0, i, 0))
           for _ in range(NSLICE)] + [
            pl.BlockSpec((2 * D, D), full),
            pl.BlockSpec((1, D), full),
            pl.BlockSpec((D, D), full),
            pl.BlockSpec((1, D), full),
            pl.BlockSpec((D, D), full),
            pl.BlockSpec((1, D), full),
        ],
        out_specs=[
            pl.BlockSpec((BN, D), lambda i: (i, 0)),
            pl.BlockSpec((BN, ED), lambda i: (i, 0)),
        ],
        out_shape=[
            jax.ShapeDtypeStruct((N, D), jnp.float32),
            jax.ShapeDtypeStruct((N, ED), jnp.float32),
        ],
    )(h, pos_pad, *ACCN, *ACCP, Wn1, bn1, Wn2, bn2, Wn3, bn3)


def kernel(h, edge_index, edge_attr, pos, We1, be1, We2, be2, We3, be3,
           Wc1, bc1, Wc2, bc2, Wc3, Wn1, bn1, Wn2, bn2, Wn3, bn3):
    rows2d = edge_index[0].reshape(NCHUNK, CHUNK)
    cols2d = edge_index[1].reshape(NCHUNK, CHUNK)
    eaT = jnp.pad(edge_attr.T, ((0, 0), (0, E2 - E)))
    We1a = We1[:D]
    We1b = We1[D:2 * D]
    We1c = We1[2 * D:2 * D + ED]
    we1d = We1[2 * D + ED:]
    pos_pad = jnp.pad(pos, ((0, 0), (0, ED - 3)))

    tabR, tabC = _prep(h, pos_pad, We1a, We1b)
    sups = [312, 313] if NSLICE == 2 else [156, 156, 156, 157]
    ew = [be1.reshape(1, D), We2, be2.reshape(1, D), We3, be3.reshape(1, D),
          Wc1, bc1.reshape(1, H), Wc2, bc2.reshape(1, H), Wc3.reshape(1, H)]
    zN = jnp.zeros((RPT, D), jnp.float32)
    zP = jnp.zeros((RPT, ED), jnp.float32)
    gs = []
    s0 = 0
    for ns in sups:
        gs.append((s0, ns, _gather(tabR, tabC, rows2d, cols2d, s0, ns)))
        s0 += ns
    es = []
    for s0, ns, (PREi, PDi, D2i) in gs:
        EOi, SCi = _edge(PREi, D2i, eaT, We1c, we1d, *ew,
                         boff=s0 * SK * CHUNK // 2048)
        es.append((s0, ns, PDi, EOi, SCi))
    accs = []
    for s0, ns, PDi, EOi, SCi in es:
        accs.append(_scatter(EOi, PDi, SCi, rows2d, zN, zP,
                             s0 * SK, ns * SK))
    ACCN = [a[0] for a in accs]
    ACCP = [a[1] for a in accs]
    h_new, pos_out = _node(h, pos_pad, ACCN, ACCP,
                           Wn1, bn1.reshape(1, D), Wn2, bn2.reshape(1, D),
                           Wn3, bn3.reshape(1, D))
    return h_new, pos_out[:, :3]


# speedup vs baseline: 1.4452x; 1.0026x over previous
import functools

import jax
import jax.numpy as jnp
from jax import lax
from jax.experimental import pallas as pl
from jax.experimental.pallas import tpu as pltpu
from jax.experimental.pallas import tpu_sc as plsc

N, E, D, H, ED = 10000, 320000, 128, 64, 16
E2 = 327680
EPS = 1e-08
W = D + ED
NC, NS, L = 2, 16, 16
NW = NC * NS
CHUNK = 128
NCHUNK = E // CHUNK
SK = 4
NSUPER = NCHUNK // SK
SKS = 2
NSUPER_S = NCHUNK // SKS
RPT = N // NS

_SC_PARAMS = pltpu.CompilerParams(use_tc_tiling_on_sc=False,
                                  needs_layout_passes=False)


def _silu(x):
    return x * jax.nn.sigmoid(x)


def _dot(a, b):
    return jnp.dot(a, b, preferred_element_type=jnp.float32)


def _prep_body(h_ref, pp_ref, wa_ref, wb_ref, outR_ref, outC_ref):
    hr = h_ref[...]
    pp = pp_ref[...]
    outR_ref[...] = jnp.concatenate([_dot(hr, wa_ref[...]), pp], axis=1)
    outC_ref[...] = jnp.concatenate([_dot(hr, wb_ref[...]), -pp], axis=1)


def _prep(h, pos_pad, We1a, We1b):
    BN = 2000
    return pl.pallas_call(
        _prep_body,
        grid=(N // BN,),
        in_specs=[
            pl.BlockSpec((BN, D), lambda i: (i, 0)),
            pl.BlockSpec((BN, ED), lambda i: (i, 0)),
            pl.BlockSpec((D, D), lambda i: (0, 0)),
            pl.BlockSpec((D, D), lambda i: (0, 0)),
        ],
        out_specs=[
            pl.BlockSpec((BN, W), lambda i: (i, 0)),
            pl.BlockSpec((BN, W), lambda i: (i, 0)),
        ],
        out_shape=[
            jax.ShapeDtypeStruct((N, W), jnp.float32),
            jax.ShapeDtypeStruct((N, W), jnp.float32),
        ],
    )(h, pos_pad, We1a, We1b)


def _gather(tabR, tabC, rows2d, cols2d, sup0, nsup):
    chunk0 = sup0 * SK
    nch = nsup * SK
    E_h = nch * CHUNK
    E2_h = ((E_h + 2047) // 2048) * 2048
    mesh = plsc.VectorSubcoreMesh(core_axis_name="c", subcore_axis_name="s")

    @functools.partial(
        pl.kernel,
        out_type=[
            jax.ShapeDtypeStruct((E2_h, D), jnp.float32),
            jax.ShapeDtypeStruct((E_h, ED), jnp.float32),
            jax.ShapeDtypeStruct((E2_h,), jnp.float32),
        ],
        mesh=mesh,
        scratch_types=[
            pltpu.VMEM((SK, CHUNK), jnp.int32),
            pltpu.VMEM((SK, CHUNK), jnp.int32),
            pltpu.VMEM((SK, CHUNK, W), jnp.float32),
            pltpu.VMEM((SK, CHUNK), jnp.float32),
            pltpu.SemaphoreType.DMA,
            pltpu.SemaphoreType.DMA,
        ] + [pltpu.SemaphoreType.DMA] * (2 * SK),
        compiler_params=_SC_PARAMS,
    )
    def k(tabR_h, tabC_h, rows_h, cols_h, pre_h, pd_h, d2_h,
          idxr, idxc, buf, d2b, semI, semW, *sems):
        semR = sems[:SK]
        semC = sems[SK:2 * SK]
        c = lax.axis_index("c")
        s = lax.axis_index("s")
        wid = s * NC + c
        nsuper = nsup // NW + jnp.where(wid < nsup % NW, 1, 0)
        lane = lax.iota(jnp.int32, L)

        @pl.loop(0, nsuper)
        def _(i):
            base = (wid + i * NW) * SK
            fires = [
                pltpu.async_copy(rows_h.at[pl.ds(chunk0 + base, SK)],
                                 idxr, semI),
                pltpu.async_copy(cols_h.at[pl.ds(chunk0 + base, SK)],
                                 idxc, semI),
            ]
            for dsc in fires:
                dsc.wait()
            rf = [pltpu.async_copy(tabR_h.at[idxr.at[j]], buf.at[j], semR[j])
                  for j in range(SK)]
            cf = []
            for j in range(SK):
                rf[j].wait()
                cf.append(pltpu.async_copy(tabC_h.at[idxc.at[j]], buf.at[j],
                                           semC[j], add=True))
            fires = []
            for j in range(SK):
                kk = base + j
                cf[j].wait()
                fires.append(pltpu.async_copy(
                    buf.at[j, :, pl.ds(0, D)],
                    pre_h.at[pl.ds(kk * CHUNK, CHUNK)], semW))
                fires.append(pltpu.async_copy(
                    buf.at[j, :, pl.ds(D, ED)],
                    pd_h.at[pl.ds(kk * CHUNK, CHUNK)], semW))
                for g in range(CHUNK // L):
                    ids = g * L + lane
                    dx = plsc.load_gather(buf.at[j],
                                          [ids, jnp.full((L,), D, jnp.int32)])
                    dy = plsc.load_gather(buf.at[j],
                                          [ids, jnp.full((L,), D + 1, jnp.int32)])
                    dz = plsc.load_gather(buf.at[j],
                                          [ids, jnp.full((L,), D + 2, jnp.int32)])
                    d2b[j, pl.ds(g * L, L)] = dx * dx + dy * dy + dz * dz
                fires.append(pltpu.async_copy(
                    d2b.at[j], d2_h.at[pl.ds(kk * CHUNK, CHUNK)], semW))
            for dsc in fires:
                dsc.wait()

    return k(tabR, tabC, rows2d, cols2d)


def _edge_body(pre_ref, d2_ref, eaT_ref, we1c_ref, we1d_ref, be1_ref,
               we2_ref, be2_ref, we3_ref, be3_ref, wc1_ref, bc1_ref,
               wc2_ref, bc2_ref, wc3_ref, eo_ref, scal_ref):
    dist2 = jnp.clip(jnp.transpose(d2_ref[...][None, :], (1, 0)), EPS, 100.0)
    norm = jnp.sqrt(dist2) + EPS
    ea_term = lax.dot_general(eaT_ref[...], we1c_ref[...],
                              (((0,), (0,)), ((), ())),
                              preferred_element_type=jnp.float32)
    a1 = pre_ref[...] + ea_term + dist2 * we1d_ref[...] + be1_ref[...]
    e1 = _silu(a1)
    e2 = _silu(_dot(e1, we2_ref[...]) + be2_ref[...])
    e = _dot(e2, we3_ref[...]) + be3_ref[...]
    c1 = _silu(_dot(e, wc1_ref[...]) + bc1_ref[...])
    c2 = _silu(_dot(c1, wc2_ref[...]) + bc2_ref[...])
    cu = jnp.clip(jnp.sum(c2 * wc3_ref[...], axis=1, keepdims=True),
                  -1.0, 1.0)
    eo_ref[...] = e
    scal_ref[...] = jnp.transpose(cu / norm, (1, 0)).reshape(scal_ref.shape)


def _edge(PRE, D2, eaT, We1c, we1d, be1, We2, be2, We3, be3,
          Wc1, bc1, Wc2, bc2, Wc3, boff):
    BE = 2048
    E2_h = PRE.shape[0]
    full = lambda i: (0, 0)
    return pl.pallas_call(
        _edge_body,
        grid=(E2_h // BE,),
        in_specs=[
            pl.BlockSpec((BE, D), lambda i: (i, 0)),
            pl.BlockSpec((BE,), lambda i: (i,)),
            pl.BlockSpec((ED, BE), lambda i: (0, i + boff)),
            pl.BlockSpec((ED, D), full),
            pl.BlockSpec((1, D), full),
            pl.BlockSpec((1, D), full),
            pl.BlockSpec((D, D), full),
            pl.BlockSpec((1, D), full),
            pl.BlockSpec((D, D), full),
            pl.BlockSpec((1, D), full),
            pl.BlockSpec((D, H), full),
            pl.BlockSpec((1, H), full),
            pl.BlockSpec((H, H), full),
            pl.BlockSpec((1, H), full),
            pl.BlockSpec((1, H), full),
        ],
        out_specs=[
            pl.BlockSpec((BE, D), lambda i: (i, 0)),
            pl.BlockSpec((BE,), lambda i: (i,)),
        ],
        out_shape=[
            jax.ShapeDtypeStruct((E2_h, D), jnp.float32),
            jax.ShapeDtypeStruct((E2_h,), jnp.float32),
        ],
    )(PRE, D2, eaT, We1c, we1d, be1, We2, be2, We3, be3,
      Wc1, bc1, Wc2, bc2, Wc3)


def _scatter(EOe, PD, SCAL, rows2d, zN, zP, chunk0, nch):
    nsup_s = nch // SKS
    mesh = plsc.VectorSubcoreMesh(core_axis_name="c", subcore_axis_name="s")

    @functools.partial(
        pl.kernel,
        out_type=[
            jax.ShapeDtypeStruct((NC, N, D), jnp.float32),
            jax.ShapeDtypeStruct((NC, N, ED), jnp.float32),
        ],
        mesh=mesh,
        scratch_types=[
            pltpu.VMEM((SKS, CHUNK), jnp.int32),
            pltpu.VMEM((SKS, CHUNK, D), jnp.float32),
            pltpu.VMEM((SKS, CHUNK, ED), jnp.float32),
            pltpu.VMEM((SKS, CHUNK), jnp.float32),
            pltpu.VMEM_SHARED((N, D), jnp.float32),
            pltpu.VMEM_SHARED((N, ED), jnp.float32),
            pltpu.SemaphoreType.DMA,
        ] + [pltpu.SemaphoreType.DMA] * SKS,
        compiler_params=_SC_PARAMS,
    )
    def k(eo_h, pd_h, scal_h, rows_h, zN_h, zP_h, outN_h, outP_h,
          idx, bufE, bufpos, sbuf, accN, accP, semS, *semR):
        c = lax.axis_index("c")
        s = lax.axis_index("s")
        wid = s * NC + c
        pltpu.sync_copy(zN_h, accN.at[pl.ds(s * RPT, RPT)])
        pltpu.sync_copy(zP_h, accP.at[pl.ds(s * RPT, RPT)])
        plsc.subcore_barrier()
        nsuper = nsup_s // NW + jnp.where(wid < nsup_s % NW, 1, 0)
        lane = lax.iota(jnp.int32, L)

        @pl.loop(0, nsuper)
        def _(i):
            base = (wid + i * NW) * SKS
            rf = []
            for j in range(SKS):
                kk = base + j
                rf.append([
                    pltpu.async_copy(eo_h.at[pl.ds(kk * CHUNK, CHUNK)],
                                     bufE.at[j], semR[j]),
                    pltpu.async_copy(pd_h.at[pl.ds(kk * CHUNK, CHUNK)],
                                     bufpos.at[j], semR[j]),
                    pltpu.async_copy(scal_h.at[pl.ds(kk * CHUNK, CHUNK)],
                                     sbuf.at[j], semR[j]),
                ])
            rf[0].append(pltpu.async_copy(
                rows_h.at[pl.ds(chunk0 + base, SKS)], idx, semR[0]))
            fires = []
            for j in range(SKS):
                for dsc in rf[j]:
                    dsc.wait()
                for g in range(CHUNK // L):
                    ids = g * L + lane
                    sc = sbuf[j, pl.ds(g * L, L)]
                    for comp in range(3):
                        cid = jnp.full((L,), comp, jnp.int32)
                        v = plsc.load_gather(bufpos.at[j], [ids, cid]) * sc
                        plsc.store_scatter(bufpos.at[j], [ids, cid], v)
                fires.append(pltpu.async_copy(
                    bufE.at[j], accN.at[idx.at[j]], semS, add=True))
                fires.append(pltpu.async_copy(
                    bufpos.at[j], accP.at[idx.at[j]], semS, add=True))
            for dsc in fires:
                dsc.wait()

        plsc.subcore_barrier()
        pltpu.sync_copy(accN.at[pl.ds(s * RPT, RPT)],
                        outN_h.at[c, pl.ds(s * RPT, RPT)])
        pltpu.sync_copy(accP.at[pl.ds(s * RPT, RPT)],
                        outP_h.at[c, pl.ds(s * RPT, RPT)])

    return k(EOe, PD, SCAL, rows2d, zN, zP)


NSLICE = 4


def _node_body(h_ref, pp_ref, *refs):
    ans = refs[:NSLICE]
    aps = refs[NSLICE:2 * NSLICE]
    (wn1_ref, bn1_ref, wn2_ref, bn2_ref, wn3_ref, bn3_ref,
     hout_ref, pout_ref) = refs[2 * NSLICE:]
    aggn = sum((a[0] + a[1] for a in ans[1:]), ans[0][0] + ans[0][1])
    x = jnp.concatenate([h_ref[...], aggn], axis=1)
    n1 = _silu(_dot(x, wn1_ref[...]) + bn1_ref[...])
    n2 = _silu(_dot(n1, wn2_ref[...]) + bn2_ref[...])
    hn = _dot(n2, wn3_ref[...]) + bn3_ref[...]
    hout_ref[...] = h_ref[...] + hn
    pout_ref[...] = pp_ref[...] + sum(
        (a[0] + a[1] for a in aps[1:]), aps[0][0] + aps[0][1])


def _node(h, pos_pad, ACCN, ACCP, Wn1, bn1, Wn2, bn2, Wn3, bn3):
    BN = 2000
    full = lambda i: (0, 0)
    return pl.pallas_call(
        _node_body,
        grid=(N // BN,),
        in_specs=[
            pl.BlockSpec((BN, D), lambda i: (i, 0)),
            pl.BlockSpec((BN, ED), lambda i: (i, 0)),
        ] + [pl.BlockSpec((NC, BN, D), lambda i: (0, i, 0))
             for _ in range(NSLICE)]
        + [pl.BlockSpec((NC, BN, ED), lambda i: (0, i, 0))
           for _ in range(NSLICE)] + [
            pl.BlockSpec((2 * D, D), full),
            pl.BlockSpec((1, D), full),
            pl.BlockSpec((D, D), full),
            pl.BlockSpec((1, D), full),
            pl.BlockSpec((D, D), full),
            pl.BlockSpec((1, D), full),
        ],
        out_specs=[
            pl.BlockSpec((BN, D), lambda i: (i, 0)),
            pl.BlockSpec((BN, ED), lambda i: (i, 0)),
        ],
        out_shape=[
            jax.ShapeDtypeStruct((N, D), jnp.float32),
            jax.ShapeDtypeStruct((N, ED), jnp.float32),
        ],
    )(h, pos_pad, *ACCN, *ACCP, Wn1, bn1, Wn2, bn2, Wn3, bn3)


def kernel(h, edge_index, edge_attr, pos, We1, be1, We2, be2, We3, be3,
           Wc1, bc1, Wc2, bc2, Wc3, Wn1, bn1, Wn2, bn2, Wn3, bn3):
    rows2d = edge_index[0].reshape(NCHUNK, CHUNK)
    cols2d = edge_index[1].reshape(NCHUNK, CHUNK)
    eaT = jnp.pad(edge_attr.T, ((0, 0), (0, E2 - E)))
    We1a = We1[:D]
    We1b = We1[D:2 * D]
    We1c = We1[2 * D:2 * D + ED]
    we1d = We1[2 * D + ED:]
    pos_pad = jnp.pad(pos, ((0, 0), (0, ED - 3)))

    tabR, tabC = _prep(h, pos_pad, We1a, We1b)
    sups = [312, 313] if NSLICE == 2 else [156, 156, 156, 157]
    ew = [be1.reshape(1, D), We2, be2.reshape(1, D), We3, be3.reshape(1, D),
          Wc1, bc1.reshape(1, H), Wc2, bc2.reshape(1, H), Wc3.reshape(1, H)]
    zN = jnp.zeros((RPT, D), jnp.float32)
    zP = jnp.zeros((RPT, ED), jnp.float32)
    gs = []
    s0 = 0
    for ns in sups:
        gs.append((s0, ns, _gather(tabR, tabC, rows2d, cols2d, s0, ns)))
        s0 += ns
    es = []
    for s0, ns, (PREi, PDi, D2i) in gs:
        EOi, SCi = _edge(PREi, D2i, eaT, We1c, we1d, *ew,
                         boff=s0 * SK * CHUNK // 2048)
        es.append((s0, ns, PDi, EOi, SCi))
    accs = []
    for s0, ns, PDi, EOi, SCi in es:
        accs.append(_scatter(EOi, PDi, SCi, rows2d, zN, zP,
                             s0 * SK, ns * SK))
    ACCN = [a[0] for a in accs]
    ACCP = [a[1] for a in accs]
    h_new, pos_out = _node(h, pos_pad, ACCN, ACCP,
                           Wn1, bn1.reshape(1, D), Wn2, bn2.reshape(1, D),
                           Wn3, bn3.reshape(1, D))
    return h_new, pos_out[:, :3]
